# baseline (speedup 1.0000x reference)
import jax
import jax.numpy as jnp
from jax import lax
from jax.experimental import pallas as pl
from jax.experimental.pallas import tpu as pltpu
from jax.experimental.pallas import tpu_sc as plsc

N_NODES = 100000
DEG = 16
D = 128
B = 16384
EMB = 128
CH_W = (1.0, 0.5)

NC = 2
NS = 16
NW = NC * NS
PER_W = B // NW
CCH = 8
NCH = PER_W // CCH
CSELF = 64
NSELF = PER_W // CSELF
G16 = PER_W // 16
CROWS = CCH * DEG
NSLOT = 4


def _sc_gather_body(nodes_hbm, feat_hbm, nidx0_hbm, nidx1_hbm,
                    self_hbm, sum0_hbm, sum1_hbm,
                    nodes_v, blkA_v, blkB_v, flat0_v, flat1_v,
                    rows_v, zeros_v, dstidx_v, selfb_v, acc_sh,
                    semA, semB, semF, semS):
    wid = lax.axis_index("s") * NC + lax.axis_index("c")
    sid = lax.axis_index("s")
    base = wid * PER_W
    pltpu.sync_copy(nodes_hbm.at[pl.ds(base, PER_W)], nodes_v)

    for c in range(CCH):
        for g in range(D // 16):
            zeros_v[c, pl.ds(g * 16, 16)] = jnp.zeros((16,), jnp.float32)
        dstidx_v[pl.ds(c * 16, 16)] = jnp.full((16,), c, jnp.int32)

    def self_chunk(j, carry):
        off = j * CSELF
        pltpu.async_copy(
            feat_hbm.at[nodes_v.at[pl.ds(off, CSELF)]], selfb_v, semS).wait()
        pltpu.sync_copy(selfb_v, self_hbm.at[pl.ds(base + off, CSELF)])
        return carry

    lax.fori_loop(0, NSELF, self_chunk, 0)

    def fetch_idx(nidx_hbm, flat_v):
        def fire_grp(g, blk_v, sem):
            nvec = nodes_v[pl.ds(g * 16, 16)]
            for l in range(16):
                blk0 = pl.multiple_of((nvec[l] >> 3) << 3, 8)
                pltpu.async_copy(
                    nidx_hbm.at[pl.ds(blk0, 8), :], blk_v.at[l], sem)

        def drain_grp(blk_v, sem):
            for l in range(16):
                pltpu.make_async_copy(
                    nidx_hbm.at[pl.ds(0, 8), :], blk_v.at[l], sem).wait()

        def extract_grp(g, blk_v):
            nvec = nodes_v[pl.ds(g * 16, 16)]
            for l in range(16):
                dst = pl.multiple_of((g * 16 + l) * DEG, 8)
                flat_v[pl.ds(dst, DEG)] = blk_v[l, nvec[l] & 7, :]

        fire_grp(0, blkA_v, semA)

        def gpair(i, carry):
            g0 = i * 2
            fire_grp(g0 + 1, blkB_v, semB)
            drain_grp(blkA_v, semA)
            extract_grp(g0, blkA_v)

            @pl.when(i < G16 // 2 - 1)
            def _():
                fire_grp(g0 + 2, blkA_v, semA)

            drain_grp(blkB_v, semB)
            extract_grp(g0 + 1, blkB_v)
            return carry

        lax.fori_loop(0, G16 // 2, gpair, 0)

    fetch_idx(nidx0_hbm, flat0_v)
    fetch_idx(nidx1_hbm, flat1_v)

    def run_channel(flat_v, out_hbm):
        def fire_rows(j, s):
            src = pl.multiple_of(j * CROWS, 8)
            pltpu.async_copy(
                feat_hbm.at[flat_v.at[pl.ds(src, CROWS)]],
                rows_v.at[s], semF[s])

        def drain_rows(s):
            pltpu.make_async_copy(
                feat_hbm.at[flat_v.at[pl.ds(0, CROWS)]],
                rows_v.at[s], semF[s]).wait()

        def add_sync(s):
            pltpu.sync_copy(rows_v.at[s], acc_sh.at[sid, s].at[dstidx_v],
                            add=True)

        def readback(j, s):
            pltpu.sync_copy(acc_sh.at[sid, s],
                            out_hbm.at[pl.ds(base + j * CCH, CCH)])

        def zero(s):
            pltpu.sync_copy(zeros_v, acc_sh.at[sid, s])

        for s in range(NSLOT):
            zero(s)
        fire_rows(0, 0)
        fire_rows(1, 1)
        fire_rows(2, 2)

        def quad(i, carry):
            for s in range(NSLOT):
                j = i * NSLOT + s
                drain_rows(s)
                add_sync(s)
                readback(j, s)
                zero(s)

                @pl.when(j + 3 < NCH)
                def _(j=j, s=s):
                    fire_rows(j + 3, (s + 3) % NSLOT)

            return carry

        lax.fori_loop(0, NCH // NSLOT, quad, 0)

    run_channel(flat0_v, sum0_hbm)
    run_channel(flat1_v, sum1_hbm)


def _sc_gather(nodes, feat_table, neigh_idx0, neigh_idx1):
    mesh = plsc.VectorSubcoreMesh(
        core_axis_name="c", subcore_axis_name="s", num_cores=NC,
        num_subcores=NS)
    f32 = jnp.float32
    i32 = jnp.int32
    return pl.kernel(
        _sc_gather_body,
        out_type=(
            jax.ShapeDtypeStruct((B, D), f32),
            jax.ShapeDtypeStruct((B, D), f32),
            jax.ShapeDtypeStruct((B, D), f32),
        ),
        mesh=mesh,
        scratch_types=[
            pltpu.VMEM((PER_W,), i32),
            pltpu.VMEM((16, 8, DEG), i32),
            pltpu.VMEM((16, 8, DEG), i32),
            pltpu.VMEM((PER_W * DEG,), i32),
            pltpu.VMEM((PER_W * DEG,), i32),
            pltpu.VMEM((NSLOT, CROWS, D), f32),
            pltpu.VMEM((CCH, D), f32),
            pltpu.VMEM((CROWS,), i32),
            pltpu.VMEM((CSELF, D), f32),
            pltpu.VMEM_SHARED((NS, NSLOT, CCH, D), f32),
            pltpu.SemaphoreType.DMA,
            pltpu.SemaphoreType.DMA,
            [pltpu.SemaphoreType.DMA] * NSLOT,
            pltpu.SemaphoreType.DMA,
        ],
    )(nodes, feat_table, neigh_idx0, neigh_idx1)


def _mlp_body(self_ref, s0_ref, s1_ref, w1_ref, b1_ref, w2_ref, b2_ref,
              out_ref):
    f32 = jnp.float32
    h = jnp.dot(self_ref[...], w1_ref[0:D, :], preferred_element_type=f32)
    h += jnp.dot(s0_ref[...] * (CH_W[0] / DEG), w1_ref[D:2 * D, :],
                 preferred_element_type=f32)
    h += jnp.dot(s1_ref[...] * (CH_W[1] / DEG), w1_ref[2 * D:3 * D, :],
                 preferred_element_type=f32)
    h = jnp.tanh(h + b1_ref[...])
    out_ref[...] = jnp.dot(h, w2_ref[...], preferred_element_type=f32) \
        + b2_ref[...]


def _mlp(self_f, sum0, sum1, W1, b1, W2, b2):
    BM = 1024
    grid = (B // BM,)
    blk = lambda i: (i, 0)
    rep = lambda i: (0, 0)
    return pl.pallas_call(
        _mlp_body,
        grid=grid,
        in_specs=[
            pl.BlockSpec((BM, D), blk),
            pl.BlockSpec((BM, D), blk),
            pl.BlockSpec((BM, D), blk),
            pl.BlockSpec((3 * D, D), rep),
            pl.BlockSpec((1, D), rep),
            pl.BlockSpec((D, EMB), rep),
            pl.BlockSpec((1, EMB), rep),
        ],
        out_specs=pl.BlockSpec((BM, EMB), blk),
        out_shape=jax.ShapeDtypeStruct((B, EMB), jnp.float32),
    )(self_f, sum0, sum1, W1, b1.reshape(1, D), W2, b2.reshape(1, EMB))


@jax.jit
def kernel(nodes, feat_table, neigh_idx0, neigh_idx1, W1, b1, W2, b2):
    self_f, sum0, sum1 = _sc_gather(nodes, feat_table, neigh_idx0,
                                    neigh_idx1)
    return _mlp(self_f, sum0, sum1, W1, b1, W2, b2)


# speedup vs baseline: 1.0126x; 1.0126x over previous
import jax
import jax.numpy as jnp
from jax import lax
from jax.experimental import pallas as pl
from jax.experimental.pallas import tpu as pltpu
from jax.experimental.pallas import tpu_sc as plsc

N_NODES = 100000
DEG = 16
D = 128
B = 16384
EMB = 128
CH_W = (1.0, 0.5)

NC = 2
NS = 16
NW = NC * NS
PER_W = B // NW
CCH = 16
NCH = PER_W // CCH
CSELF = 64
NSELF = PER_W // CSELF
G16 = PER_W // 16
CROWS = CCH * DEG


def _tree_sum16(vals):
    while len(vals) > 1:
        vals = [vals[i] + vals[i + 1] for i in range(0, len(vals) - 1, 2)] \
            + ([vals[-1]] if len(vals) % 2 else [])
    return vals[0]


def _sc_gather_body(nodes_hbm, feat_hbm, nidx0_hbm, nidx1_hbm,
                    self_hbm, sum0_hbm, sum1_hbm,
                    nodes_v, blkA_v, blkB_v, flat0_v, flat1_v,
                    rowsA_v, rowsB_v, outA_v, outB_v, selfb_v,
                    semA, semB, semFA, semFB, semOA, semOB, semS):
    wid = lax.axis_index("s") * NC + lax.axis_index("c")
    base = wid * PER_W
    pltpu.sync_copy(nodes_hbm.at[pl.ds(base, PER_W)], nodes_v)

    def self_chunk(j, carry):
        off = j * CSELF
        pltpu.async_copy(
            feat_hbm.at[nodes_v.at[pl.ds(off, CSELF)]], selfb_v, semS).wait()
        pltpu.sync_copy(selfb_v, self_hbm.at[pl.ds(base + off, CSELF)])
        return carry

    lax.fori_loop(0, NSELF, self_chunk, 0)

    def fetch_idx(nidx_hbm, flat_v):
        def fire_grp(g, blk_v, sem):
            nvec = nodes_v[pl.ds(g * 16, 16)]
            for l in range(16):
                blk0 = pl.multiple_of((nvec[l] >> 3) << 3, 8)
                pltpu.async_copy(
                    nidx_hbm.at[pl.ds(blk0, 8), :], blk_v.at[l], sem)

        def drain_grp(blk_v, sem):
            for l in range(16):
                pltpu.make_async_copy(
                    nidx_hbm.at[pl.ds(0, 8), :], blk_v.at[l], sem).wait()

        def extract_grp(g, blk_v):
            nvec = nodes_v[pl.ds(g * 16, 16)]
            for l in range(16):
                dst = pl.multiple_of((g * 16 + l) * DEG, 8)
                flat_v[pl.ds(dst, DEG)] = blk_v[l, nvec[l] & 7, :]

        fire_grp(0, blkA_v, semA)

        def gpair(i, carry):
            g0 = i * 2
            fire_grp(g0 + 1, blkB_v, semB)
            drain_grp(blkA_v, semA)
            extract_grp(g0, blkA_v)

            @pl.when(i < G16 // 2 - 1)
            def _():
                fire_grp(g0 + 2, blkA_v, semA)

            drain_grp(blkB_v, semB)
            extract_grp(g0 + 1, blkB_v)
            return carry

        lax.fori_loop(0, G16 // 2, gpair, 0)

    fetch_idx(nidx0_hbm, flat0_v)
    fetch_idx(nidx1_hbm, flat1_v)

    def run_channel(flat_v, out_hbm):
        def fire(j, rows_v, semF):
            src = pl.multiple_of(j * CROWS, 8)
            pltpu.async_copy(
                feat_hbm.at[flat_v.at[pl.ds(src, CROWS)]], rows_v, semF)

        def drain_rows(rows_v, semF):
            pltpu.make_async_copy(
                feat_hbm.at[flat_v.at[pl.ds(0, CROWS)]], rows_v, semF).wait()

        def drain_out(outc_v, semO):
            pltpu.make_async_copy(
                outc_v, out_hbm.at[pl.ds(0, CCH)], semO).wait()

        def compute(j, rows_v, outc_v, semO):
            @plsc.parallel_loop(0, CCH, step=4)
            def node_sum(c):
                for u in range(4):
                    for g in range(D // 16):
                        sl = pl.ds(g * 16, 16)
                        vals = [rows_v[(c + u) * DEG + r, sl]
                                for r in range(DEG)]
                        outc_v[c + u, sl] = _tree_sum16(vals)

            pltpu.async_copy(outc_v, out_hbm.at[pl.ds(base + j * CCH, CCH)],
                             semO)

        fire(0, rowsA_v, semFA)

        def pair(i, carry):
            j0 = i * 2
            fire(j0 + 1, rowsB_v, semFB)
            drain_rows(rowsA_v, semFA)

            @pl.when(i > 0)
            def _():
                drain_out(outA_v, semOA)

            compute(j0, rowsA_v, outA_v, semOA)

            @pl.when(i < NCH // 2 - 1)
            def _():
                fire(j0 + 2, rowsA_v, semFA)

            drain_rows(rowsB_v, semFB)

            @pl.when(i > 0)
            def _():
                drain_out(outB_v, semOB)

            compute(j0 + 1, rowsB_v, outB_v, semOB)
            return carry

        lax.fori_loop(0, NCH // 2, pair, 0)
        drain_out(outA_v, semOA)
        drain_out(outB_v, semOB)

    run_channel(flat0_v, sum0_hbm)
    run_channel(flat1_v, sum1_hbm)


def _sc_gather(nodes, feat_table, neigh_idx0, neigh_idx1):
    mesh = plsc.VectorSubcoreMesh(
        core_axis_name="c", subcore_axis_name="s", num_cores=NC,
        num_subcores=NS)
    f32 = jnp.float32
    i32 = jnp.int32
    return pl.kernel(
        _sc_gather_body,
        out_type=(
            jax.ShapeDtypeStruct((B, D), f32),
            jax.ShapeDtypeStruct((B, D), f32),
            jax.ShapeDtypeStruct((B, D), f32),
        ),
        mesh=mesh,
        scratch_types=[
            pltpu.VMEM((PER_W,), i32),
            pltpu.VMEM((16, 8, DEG), i32),
            pltpu.VMEM((16, 8, DEG), i32),
            pltpu.VMEM((PER_W * DEG,), i32),
            pltpu.VMEM((PER_W * DEG,), i32),
            pltpu.VMEM((CROWS, D), f32),
            pltpu.VMEM((CROWS, D), f32),
            pltpu.VMEM((CCH, D), f32),
            pltpu.VMEM((CCH, D), f32),
            pltpu.VMEM((CSELF, D), f32),
            pltpu.SemaphoreType.DMA,
            pltpu.SemaphoreType.DMA,
            pltpu.SemaphoreType.DMA,
            pltpu.SemaphoreType.DMA,
            pltpu.SemaphoreType.DMA,
            pltpu.SemaphoreType.DMA,
            pltpu.SemaphoreType.DMA,
        ],
    )(nodes, feat_table, neigh_idx0, neigh_idx1)


def _mlp_body(self_ref, s0_ref, s1_ref, w1_ref, b1_ref, w2_ref, b2_ref,
              out_ref):
    f32 = jnp.float32
    h = jnp.dot(self_ref[...], w1_ref[0:D, :], preferred_element_type=f32)
    h += jnp.dot(s0_ref[...] * (CH_W[0] / DEG), w1_ref[D:2 * D, :],
                 preferred_element_type=f32)
    h += jnp.dot(s1_ref[...] * (CH_W[1] / DEG), w1_ref[2 * D:3 * D, :],
                 preferred_element_type=f32)
    h = jnp.tanh(h + b1_ref[...])
    out_ref[...] = jnp.dot(h, w2_ref[...], preferred_element_type=f32) \
        + b2_ref[...]


def _mlp(self_f, sum0, sum1, W1, b1, W2, b2):
    BM = 1024
    grid = (B // BM,)
    blk = lambda i: (i, 0)
    rep = lambda i: (0, 0)
    return pl.pallas_call(
        _mlp_body,
        grid=grid,
        in_specs=[
            pl.BlockSpec((BM, D), blk),
            pl.BlockSpec((BM, D), blk),
            pl.BlockSpec((BM, D), blk),
            pl.BlockSpec((3 * D, D), rep),
            pl.BlockSpec((1, D), rep),
            pl.BlockSpec((D, EMB), rep),
            pl.BlockSpec((1, EMB), rep),
        ],
        out_specs=pl.BlockSpec((BM, EMB), blk),
        out_shape=jax.ShapeDtypeStruct((B, EMB), jnp.float32),
    )(self_f, sum0, sum1, W1, b1.reshape(1, D), W2, b2.reshape(1, EMB))


@jax.jit
def kernel(nodes, feat_table, neigh_idx0, neigh_idx1, W1, b1, W2, b2):
    self_f, sum0, sum1 = _sc_gather(nodes, feat_table, neigh_idx0,
                                    neigh_idx1)
    return _mlp(self_f, sum0, sum1, W1, b1, W2, b2)


# speedup vs baseline: 1.0944x; 1.0807x over previous
import jax
import jax.numpy as jnp
from jax import lax
from jax.experimental import pallas as pl
from jax.experimental.pallas import tpu as pltpu
from jax.experimental.pallas import tpu_sc as plsc

N_NODES = 100000
DEG = 16
D = 128
B = 16384
EMB = 128
CH_W = (1.0, 0.5)

NC = 2
NS = 16
NW = NC * NS
PER_W = B // NW
CCH = 16
NCH = PER_W // CCH
CSELF = 64
NSELF = PER_W // CSELF
G16 = PER_W // 16
CROWS = CCH * DEG


def _tree_sum16(vals):
    while len(vals) > 1:
        vals = [vals[i] + vals[i + 1] for i in range(0, len(vals) - 1, 2)] \
            + ([vals[-1]] if len(vals) % 2 else [])
    return vals[0]


def _sc_self_body(nodes_hbm, feat_hbm, self_hbm, nodes_v, selfb_v, semS):
    wid = lax.axis_index("s") * NC + lax.axis_index("c")
    base = wid * PER_W
    pltpu.sync_copy(nodes_hbm.at[pl.ds(base, PER_W)], nodes_v)

    def self_chunk(j, carry):
        off = j * CSELF
        pltpu.async_copy(
            feat_hbm.at[nodes_v.at[pl.ds(off, CSELF)]], selfb_v, semS).wait()
        pltpu.sync_copy(selfb_v, self_hbm.at[pl.ds(base + off, CSELF)])
        return carry

    lax.fori_loop(0, NSELF, self_chunk, 0)


def _sc_gather_body(nodes_hbm, feat_hbm, nidx0_hbm, nidx1_hbm,
                    sum0_hbm, sum1_hbm,
                    nodes_v, blkA_v, blkB_v, flat0_v, flat1_v,
                    rowsA_v, rowsB_v, outA_v, outB_v,
                    semA, semB, semFA, semFB, semOA, semOB):
    wid = lax.axis_index("s") * NC + lax.axis_index("c")
    base = wid * PER_W
    pltpu.sync_copy(nodes_hbm.at[pl.ds(base, PER_W)], nodes_v)

    def fetch_idx(nidx_hbm, flat_v):
        def fire_grp(g, blk_v, sem):
            nvec = nodes_v[pl.ds(g * 16, 16)]
            for l in range(16):
                blk0 = pl.multiple_of((nvec[l] >> 3) << 3, 8)
                pltpu.async_copy(
                    nidx_hbm.at[pl.ds(blk0, 8), :], blk_v.at[l], sem)

        def drain_grp(blk_v, sem):
            for l in range(16):
                pltpu.make_async_copy(
                    nidx_hbm.at[pl.ds(0, 8), :], blk_v.at[l], sem).wait()

        def extract_grp(g, blk_v):
            nvec = nodes_v[pl.ds(g * 16, 16)]
            for l in range(16):
                dst = pl.multiple_of((g * 16 + l) * DEG, 8)
                flat_v[pl.ds(dst, DEG)] = blk_v[l, nvec[l] & 7, :]

        fire_grp(0, blkA_v, semA)

        def gpair(i, carry):
            g0 = i * 2
            fire_grp(g0 + 1, blkB_v, semB)
            drain_grp(blkA_v, semA)
            extract_grp(g0, blkA_v)

            @pl.when(i < G16 // 2 - 1)
            def _():
                fire_grp(g0 + 2, blkA_v, semA)

            drain_grp(blkB_v, semB)
            extract_grp(g0 + 1, blkB_v)
            return carry

        lax.fori_loop(0, G16 // 2, gpair, 0)

    fetch_idx(nidx0_hbm, flat0_v)
    fetch_idx(nidx1_hbm, flat1_v)

    def run_channel(flat_v, out_hbm):
        def fire(j, rows_v, semF):
            src = pl.multiple_of(j * CROWS, 8)
            pltpu.async_copy(
                feat_hbm.at[flat_v.at[pl.ds(src, CROWS)]], rows_v, semF)

        def drain_rows(rows_v, semF):
            pltpu.make_async_copy(
                feat_hbm.at[flat_v.at[pl.ds(0, CROWS)]], rows_v, semF).wait()

        def drain_out(outc_v, semO):
            pltpu.make_async_copy(
                outc_v, out_hbm.at[pl.ds(0, CCH)], semO).wait()

        def compute(j, rows_v, outc_v, semO):
            @plsc.parallel_loop(0, CCH, step=2)
            def node_sum(c):
                for u in range(2):
                    for g in range(D // 16):
                        sl = pl.ds(g * 16, 16)
                        vals = [rows_v[(c + u) * DEG + r, sl]
                                for r in range(DEG)]
                        outc_v[c + u, sl] = _tree_sum16(vals)

            pltpu.async_copy(outc_v, out_hbm.at[pl.ds(base + j * CCH, CCH)],
                             semO)

        fire(0, rowsA_v, semFA)

        def pair(i, carry):
            j0 = i * 2
            fire(j0 + 1, rowsB_v, semFB)
            drain_rows(rowsA_v, semFA)

            @pl.when(i > 0)
            def _():
                drain_out(outA_v, semOA)

            compute(j0, rowsA_v, outA_v, semOA)

            @pl.when(i < NCH // 2 - 1)
            def _():
                fire(j0 + 2, rowsA_v, semFA)

            drain_rows(rowsB_v, semFB)

            @pl.when(i > 0)
            def _():
                drain_out(outB_v, semOB)

            compute(j0 + 1, rowsB_v, outB_v, semOB)
            return carry

        lax.fori_loop(0, NCH // 2, pair, 0)
        drain_out(outA_v, semOA)
        drain_out(outB_v, semOB)

    run_channel(flat0_v, sum0_hbm)
    run_channel(flat1_v, sum1_hbm)


def _sc_gather(nodes, feat_table, neigh_idx0, neigh_idx1):
    mesh = plsc.VectorSubcoreMesh(
        core_axis_name="c", subcore_axis_name="s", num_cores=NC,
        num_subcores=NS)
    f32 = jnp.float32
    i32 = jnp.int32
    self_f = pl.kernel(
        _sc_self_body,
        out_type=jax.ShapeDtypeStruct((B, D), f32),
        mesh=mesh,
        scratch_types=[
            pltpu.VMEM((PER_W,), i32),
            pltpu.VMEM((CSELF, D), f32),
            pltpu.SemaphoreType.DMA,
        ],
    )(nodes, feat_table)
    sum0, sum1 = pl.kernel(
        _sc_gather_body,
        out_type=(
            jax.ShapeDtypeStruct((B, D), f32),
            jax.ShapeDtypeStruct((B, D), f32),
        ),
        mesh=mesh,
        scratch_types=[
            pltpu.VMEM((PER_W,), i32),
            pltpu.VMEM((16, 8, DEG), i32),
            pltpu.VMEM((16, 8, DEG), i32),
            pltpu.VMEM((PER_W * DEG,), i32),
            pltpu.VMEM((PER_W * DEG,), i32),
            pltpu.VMEM((CROWS, D), f32),
            pltpu.VMEM((CROWS, D), f32),
            pltpu.VMEM((CCH, D), f32),
            pltpu.VMEM((CCH, D), f32),
            pltpu.SemaphoreType.DMA,
            pltpu.SemaphoreType.DMA,
            pltpu.SemaphoreType.DMA,
            pltpu.SemaphoreType.DMA,
            pltpu.SemaphoreType.DMA,
            pltpu.SemaphoreType.DMA,
        ],
    )(nodes, feat_table, neigh_idx0, neigh_idx1)
    return self_f, sum0, sum1


def _mlp_body(self_ref, s0_ref, s1_ref, w1_ref, b1_ref, w2_ref, b2_ref,
              out_ref):
    f32 = jnp.float32
    h = jnp.dot(self_ref[...], w1_ref[0:D, :], preferred_element_type=f32)
    h += jnp.dot(s0_ref[...] * (CH_W[0] / DEG), w1_ref[D:2 * D, :],
                 preferred_element_type=f32)
    h += jnp.dot(s1_ref[...] * (CH_W[1] / DEG), w1_ref[2 * D:3 * D, :],
                 preferred_element_type=f32)
    h = jnp.tanh(h + b1_ref[...])
    out_ref[...] = jnp.dot(h, w2_ref[...], preferred_element_type=f32) \
        + b2_ref[...]


def _mlp(self_f, sum0, sum1, W1, b1, W2, b2):
    BM = 1024
    grid = (B // BM,)
    blk = lambda i: (i, 0)
    rep = lambda i: (0, 0)
    return pl.pallas_call(
        _mlp_body,
        grid=grid,
        in_specs=[
            pl.BlockSpec((BM, D), blk),
            pl.BlockSpec((BM, D), blk),
            pl.BlockSpec((BM, D), blk),
            pl.BlockSpec((3 * D, D), rep),
            pl.BlockSpec((1, D), rep),
            pl.BlockSpec((D, EMB), rep),
            pl.BlockSpec((1, EMB), rep),
        ],
        out_specs=pl.BlockSpec((BM, EMB), blk),
        out_shape=jax.ShapeDtypeStruct((B, EMB), jnp.float32),
    )(self_f, sum0, sum1, W1, b1.reshape(1, D), W2, b2.reshape(1, EMB))


@jax.jit
def kernel(nodes, feat_table, neigh_idx0, neigh_idx1, W1, b1, W2, b2):
    self_f, sum0, sum1 = _sc_gather(nodes, feat_table, neigh_idx0,
                                    neigh_idx1)
    return _mlp(self_f, sum0, sum1, W1, b1, W2, b2)


# speedup vs baseline: 1.1100x; 1.0142x over previous
import jax
import jax.numpy as jnp
from jax import lax
from jax.experimental import pallas as pl
from jax.experimental.pallas import tpu as pltpu
from jax.experimental.pallas import tpu_sc as plsc

N_NODES = 100000
DEG = 16
D = 128
B = 16384
EMB = 128
CH_W = (1.0, 0.5)

NC = 2
NS = 16
NW = NC * NS
PER_W = B // NW
CCH = 16
NCH = PER_W // CCH
CSELF = 64
NSELF = PER_W // CSELF
G16 = PER_W // 16
CROWS = CCH * DEG


def _tree_sum16(vals):
    while len(vals) > 1:
        vals = [vals[i] + vals[i + 1] for i in range(0, len(vals) - 1, 2)] \
            + ([vals[-1]] if len(vals) % 2 else [])
    return vals[0]


def _sc_gather_body(nodes_hbm, feat_hbm, nidx0_hbm, nidx1_hbm,
                    self_hbm, sum0_hbm, sum1_hbm,
                    nodes_v, blkA_v, blkB_v, flat0_v, flat1_v,
                    rowsA_v, rowsB_v, outA_v, outB_v, selfb_v,
                    semA, semB, semFA, semFB, semOA, semOB, semS):
    wid = lax.axis_index("s") * NC + lax.axis_index("c")
    base = wid * PER_W
    pltpu.sync_copy(nodes_hbm.at[pl.ds(base, PER_W)], nodes_v)

    def self_chunk(j, carry):
        off = j * CSELF
        pltpu.async_copy(
            feat_hbm.at[nodes_v.at[pl.ds(off, CSELF)]], selfb_v, semS).wait()
        pltpu.sync_copy(selfb_v, self_hbm.at[pl.ds(base + off, CSELF)])
        return carry

    lax.fori_loop(0, NSELF, self_chunk, 0)

    def fetch_idx(nidx_hbm, flat_v):
        def fire_grp(g, blk_v, sem):
            nvec = nodes_v[pl.ds(g * 16, 16)]
            for l in range(16):
                blk0 = pl.multiple_of((nvec[l] >> 3) << 3, 8)
                pltpu.async_copy(
                    nidx_hbm.at[pl.ds(blk0, 8), :], blk_v.at[l], sem)

        def drain_grp(blk_v, sem):
            for l in range(16):
                pltpu.make_async_copy(
                    nidx_hbm.at[pl.ds(0, 8), :], blk_v.at[l], sem).wait()

        def extract_grp(g, blk_v):
            nvec = nodes_v[pl.ds(g * 16, 16)]
            for l in range(16):
                dst = pl.multiple_of((g * 16 + l) * DEG, 8)
                flat_v[pl.ds(dst, DEG)] = blk_v[l, nvec[l] & 7, :]

        fire_grp(0, blkA_v, semA)

        def gpair(i, carry):
            g0 = i * 2
            fire_grp(g0 + 1, blkB_v, semB)
            drain_grp(blkA_v, semA)
            extract_grp(g0, blkA_v)

            @pl.when(i < G16 // 2 - 1)
            def _():
                fire_grp(g0 + 2, blkA_v, semA)

            drain_grp(blkB_v, semB)
            extract_grp(g0 + 1, blkB_v)
            return carry

        lax.fori_loop(0, G16 // 2, gpair, 0)

    fetch_idx(nidx0_hbm, flat0_v)
    fetch_idx(nidx1_hbm, flat1_v)

    def run_channel(flat_v, out_hbm):
        def fire(j, rows_v, semF):
            src = pl.multiple_of(j * CROWS, 8)
            pltpu.async_copy(
                feat_hbm.at[flat_v.at[pl.ds(src, CROWS)]], rows_v, semF)

        def drain_rows(rows_v, semF):
            pltpu.make_async_copy(
                feat_hbm.at[flat_v.at[pl.ds(0, CROWS)]], rows_v, semF).wait()

        def drain_out(outc_v, semO):
            pltpu.make_async_copy(
                outc_v, out_hbm.at[pl.ds(0, CCH)], semO).wait()

        def compute(j, rows_v, outc_v, semO):
            @plsc.parallel_loop(0, CCH, step=2)
            def node_sum(c):
                for u in range(2):
                    for g in range(D // 16):
                        sl = pl.ds(g * 16, 16)
                        vals = [rows_v[(c + u) * DEG + r, sl]
                                for r in range(DEG)]
                        outc_v[c + u, sl] = _tree_sum16(vals)

            pltpu.async_copy(outc_v, out_hbm.at[pl.ds(base + j * CCH, CCH)],
                             semO)

        fire(0, rowsA_v, semFA)

        def pair(i, carry):
            j0 = i * 2
            fire(j0 + 1, rowsB_v, semFB)
            drain_rows(rowsA_v, semFA)

            @pl.when(i > 0)
            def _():
                drain_out(outA_v, semOA)

            compute(j0, rowsA_v, outA_v, semOA)

            @pl.when(i < NCH // 2 - 1)
            def _():
                fire(j0 + 2, rowsA_v, semFA)

            drain_rows(rowsB_v, semFB)

            @pl.when(i > 0)
            def _():
                drain_out(outB_v, semOB)

            compute(j0 + 1, rowsB_v, outB_v, semOB)
            return carry

        lax.fori_loop(0, NCH // 2, pair, 0)
        drain_out(outA_v, semOA)
        drain_out(outB_v, semOB)

    run_channel(flat0_v, sum0_hbm)
    run_channel(flat1_v, sum1_hbm)


def _sc_gather(nodes, feat_table, neigh_idx0, neigh_idx1):
    mesh = plsc.VectorSubcoreMesh(
        core_axis_name="c", subcore_axis_name="s", num_cores=NC,
        num_subcores=NS)
    f32 = jnp.float32
    i32 = jnp.int32
    return pl.kernel(
        _sc_gather_body,
        out_type=(
            jax.ShapeDtypeStruct((B, D), f32),
            jax.ShapeDtypeStruct((B, D), f32),
            jax.ShapeDtypeStruct((B, D), f32),
        ),
        mesh=mesh,
        scratch_types=[
            pltpu.VMEM((PER_W,), i32),
            pltpu.VMEM((16, 8, DEG), i32),
            pltpu.VMEM((16, 8, DEG), i32),
            pltpu.VMEM((PER_W * DEG,), i32),
            pltpu.VMEM((PER_W * DEG,), i32),
            pltpu.VMEM((CROWS, D), f32),
            pltpu.VMEM((CROWS, D), f32),
            pltpu.VMEM((CCH, D), f32),
            pltpu.VMEM((CCH, D), f32),
            pltpu.VMEM((CSELF, D), f32),
            pltpu.SemaphoreType.DMA,
            pltpu.SemaphoreType.DMA,
            pltpu.SemaphoreType.DMA,
            pltpu.SemaphoreType.DMA,
            pltpu.SemaphoreType.DMA,
            pltpu.SemaphoreType.DMA,
            pltpu.SemaphoreType.DMA,
        ],
    )(nodes, feat_table, neigh_idx0, neigh_idx1)


def _mlp_body(self_ref, s0_ref, s1_ref, w1_ref, b1_ref, w2_ref, b2_ref,
              out_ref):
    f32 = jnp.float32
    h = jnp.dot(self_ref[...], w1_ref[0:D, :], preferred_element_type=f32)
    h += jnp.dot(s0_ref[...] * (CH_W[0] / DEG), w1_ref[D:2 * D, :],
                 preferred_element_type=f32)
    h += jnp.dot(s1_ref[...] * (CH_W[1] / DEG), w1_ref[2 * D:3 * D, :],
                 preferred_element_type=f32)
    h = jnp.tanh(h + b1_ref[...])
    out_ref[...] = jnp.dot(h, w2_ref[...], preferred_element_type=f32) \
        + b2_ref[...]


def _mlp(self_f, sum0, sum1, W1, b1, W2, b2):
    BM = 1024
    grid = (B // BM,)
    blk = lambda i: (i, 0)
    rep = lambda i: (0, 0)
    return pl.pallas_call(
        _mlp_body,
        grid=grid,
        in_specs=[
            pl.BlockSpec((BM, D), blk),
            pl.BlockSpec((BM, D), blk),
            pl.BlockSpec((BM, D), blk),
            pl.BlockSpec((3 * D, D), rep),
            pl.BlockSpec((1, D), rep),
            pl.BlockSpec((D, EMB), rep),
            pl.BlockSpec((1, EMB), rep),
        ],
        out_specs=pl.BlockSpec((BM, EMB), blk),
        out_shape=jax.ShapeDtypeStruct((B, EMB), jnp.float32),
    )(self_f, sum0, sum1, W1, b1.reshape(1, D), W2, b2.reshape(1, EMB))


@jax.jit
def kernel(nodes, feat_table, neigh_idx0, neigh_idx1, W1, b1, W2, b2):
    self_f, sum0, sum1 = _sc_gather(nodes, feat_table, neigh_idx0,
                                    neigh_idx1)
    return _mlp(self_f, sum0, sum1, W1, b1, W2, b2)
